# Initial kernel scaffold; baseline (speedup 1.0000x reference)
import jax
import jax.numpy as jnp
from jax.experimental import pallas as pl


def kernel(x, concepts):
    raise NotImplementedError("write your pallas kernel here")


# speedup vs baseline: 3.7338x; 3.7338x over previous
import jax
import jax.numpy as jnp
import numpy as np
from jax.experimental import pallas as pl
from jax.experimental.pallas import tpu as pltpu

_NUM_INTERVENTIONS = 256
_CONCEPT_DIM = 1024
_ROWS = 16384

_IDX = np.array([
    2, 4, 5, 7, 16, 19, 29, 31, 34, 35, 37, 44, 45, 58, 61, 63, 65, 72, 78,
    82, 83, 85, 90, 99, 101, 102, 108, 110, 111, 112, 114, 117, 121, 123,
    129, 130, 139, 142, 144, 148, 152, 155, 156, 157, 163, 167, 174, 175,
    176, 177, 178, 179, 183, 188, 189, 197, 211, 212, 240, 251, 254, 257,
    259, 263, 268, 269, 272, 277, 278, 284, 291, 300, 302, 304, 305, 309,
    312, 315, 318, 323, 325, 336, 339, 350, 356, 363, 366, 367, 369, 379,
    388, 398, 409, 410, 415, 417, 429, 436, 441, 444, 446, 447, 448, 452,
    461, 462, 463, 480, 481, 487, 493, 495, 499, 501, 504, 507, 509, 514,
    516, 517, 518, 520, 524, 525, 532, 538, 540, 541, 542, 543, 544, 551,
    552, 553, 557, 562, 564, 565, 567, 569, 575, 577, 578, 580, 582, 584,
    585, 589, 590, 591, 598, 600, 602, 603, 605, 607, 617, 619, 638, 649,
    650, 654, 659, 670, 673, 675, 681, 690, 693, 694, 698, 703, 704, 706,
    707, 708, 709, 712, 714, 715, 730, 736, 739, 748, 750, 752, 753, 755,
    762, 765, 768, 769, 771, 774, 776, 777, 780, 787, 790, 792, 793, 799,
    803, 804, 808, 810, 816, 829, 836, 842, 846, 848, 854, 857, 859, 864,
    872, 874, 879, 883, 885, 893, 895, 901, 904, 910, 911, 914, 918, 921,
    928, 932, 934, 940, 942, 955, 957, 962, 966, 970, 973, 976, 981, 984,
    995, 996, 999, 1001, 1005, 1009, 1010, 1012, 1016, 1017, 1020, 1021,
], dtype=np.int32)
_MASK = np.zeros((8, _CONCEPT_DIM), dtype=np.float32)
_MASK[:, _IDX] = 1.0

_BLOCK_ROWS = 1024


def _select_kernel(m_ref, x_ref, c_ref, o_ref):
    mask = m_ref[0:1, :] != 0.0
    o_ref[...] = jnp.where(mask, c_ref[...], x_ref[...])


def _tc_kernel_rows(x, concepts, row0, nrows):
    grid = (nrows // _BLOCK_ROWS,)
    r0b = row0 // _BLOCK_ROWS
    in_spec = pl.BlockSpec((_BLOCK_ROWS, _CONCEPT_DIM), lambda i: (i + r0b, 0))
    out_spec = pl.BlockSpec((_BLOCK_ROWS, _CONCEPT_DIM), lambda i: (i, 0))
    mspec = pl.BlockSpec((8, _CONCEPT_DIM), lambda i: (0, 0))
    return pl.pallas_call(
        _select_kernel,
        grid=grid,
        in_specs=[mspec, in_spec, in_spec],
        out_specs=out_spec,
        out_shape=jax.ShapeDtypeStruct((nrows, _CONCEPT_DIM), jnp.float32),
        compiler_params=pltpu.CompilerParams(
            dimension_semantics=("parallel",),
        ),
    )(jnp.asarray(_MASK), x, concepts)


def _tc_kernel(x, concepts):
    return _tc_kernel_rows(x, concepts, 0, _ROWS)


import functools
from jax import lax
from jax.experimental.pallas import tpu_sc as plsc

_LANES = 16
_NW = 32
_ROWS_PER_W = _ROWS // _NW
_CHUNK = 16
_NB = 4
_NCHUNK = _ROWS_PER_W // _CHUNK
_NGROUP = _CONCEPT_DIM // _LANES

_MASK1D = np.zeros((_CONCEPT_DIM,), dtype=np.float32)
_MASK1D[_IDX] = 1.0


def _sc_body(m_hbm, x_hbm, c_hbm, o_hbm, mask_v, cbuf, obuf,
             xsem, csem, osem):
    nc = 2
    wid = lax.axis_index("s") * nc + lax.axis_index("c")
    base = wid * _ROWS_PER_W
    pltpu.sync_copy(m_hbm, mask_v)

    def start_in(ch, oslot, cslot):
        row0 = base + ch * _CHUNK
        pltpu.make_async_copy(
            x_hbm.at[pl.ds(row0, _CHUNK)], obuf.at[oslot], xsem.at[oslot]
        ).start()
        pltpu.make_async_copy(
            c_hbm.at[pl.ds(row0, _CHUNK)], cbuf.at[cslot], csem.at[cslot]
        ).start()

    def wait_in(oslot, cslot):
        pltpu.make_async_copy(
            x_hbm.at[pl.ds(base, _CHUNK)], obuf.at[oslot], xsem.at[oslot]
        ).wait()
        pltpu.make_async_copy(
            c_hbm.at[pl.ds(base, _CHUNK)], cbuf.at[cslot], csem.at[cslot]
        ).wait()

    def start_out(ch, oslot):
        row0 = base + ch * _CHUNK
        pltpu.make_async_copy(
            obuf.at[oslot], o_hbm.at[pl.ds(row0, _CHUNK)], osem.at[oslot]
        ).start()

    def wait_out(oslot):
        pltpu.make_async_copy(
            obuf.at[oslot], o_hbm.at[pl.ds(base, _CHUNK)], osem.at[oslot]
        ).wait()

    def compute(oslot, cslot):
        def g_body(g, c2):
            col = pl.multiple_of(g * _LANES, _LANES)
            mg = mask_v[pl.ds(col, _LANES)] != 0.0
            for r in range(_CHUNK):
                xv = obuf[oslot, r, pl.ds(col, _LANES)]
                cv = cbuf[cslot, r, pl.ds(col, _LANES)]
                obuf[oslot, r, pl.ds(col, _LANES)] = jnp.where(mg, cv, xv)
            return c2

        lax.fori_loop(0, _NGROUP, g_body, 0)

    start_in(0, 0, 0)

    def ring_body(j, carry):
        for b in range(_NB):
            ch = j * _NB + b
            nxt = ch + 1
            ons = (b + 1) % _NB
            cns = (b + 1) % 2

            @pl.when(nxt < _NCHUNK)
            def _():
                @pl.when(nxt >= _NB)
                def _():
                    wait_out(ons)

                start_in(nxt, ons, cns)

            wait_in(b, b % 2)
            compute(b, b % 2)
            start_out(ch, b)
        return carry

    lax.fori_loop(0, _NCHUNK // _NB, ring_body, 0)
    for b in range(_NB):
        wait_out(b)


def _sc_kernel(x, concepts):
    mesh = plsc.VectorSubcoreMesh(core_axis_name="c", subcore_axis_name="s")
    k = functools.partial(
        pl.kernel,
        mesh=mesh,
        out_type=jax.ShapeDtypeStruct((_ROWS, _CONCEPT_DIM), jnp.float32),
        scratch_types=[
            pltpu.VMEM((_CONCEPT_DIM,), jnp.float32),
            pltpu.VMEM((2, _CHUNK, _CONCEPT_DIM), jnp.float32),
            pltpu.VMEM((_NB, _CHUNK, _CONCEPT_DIM), jnp.float32),
            pltpu.SemaphoreType.DMA((_NB,)),
            pltpu.SemaphoreType.DMA((2,)),
            pltpu.SemaphoreType.DMA((_NB,)),
        ],
    )(_sc_body)
    return k(jnp.asarray(_MASK1D), x, concepts)


def kernel(x, concepts):
    return _sc_kernel(x, concepts)


# speedup vs baseline: 3.8800x; 1.0391x over previous
import jax
import jax.numpy as jnp
import numpy as np
from jax.experimental import pallas as pl
from jax.experimental.pallas import tpu as pltpu

_NUM_INTERVENTIONS = 256
_CONCEPT_DIM = 1024
_ROWS = 16384

_IDX = np.array([
    2, 4, 5, 7, 16, 19, 29, 31, 34, 35, 37, 44, 45, 58, 61, 63, 65, 72, 78,
    82, 83, 85, 90, 99, 101, 102, 108, 110, 111, 112, 114, 117, 121, 123,
    129, 130, 139, 142, 144, 148, 152, 155, 156, 157, 163, 167, 174, 175,
    176, 177, 178, 179, 183, 188, 189, 197, 211, 212, 240, 251, 254, 257,
    259, 263, 268, 269, 272, 277, 278, 284, 291, 300, 302, 304, 305, 309,
    312, 315, 318, 323, 325, 336, 339, 350, 356, 363, 366, 367, 369, 379,
    388, 398, 409, 410, 415, 417, 429, 436, 441, 444, 446, 447, 448, 452,
    461, 462, 463, 480, 481, 487, 493, 495, 499, 501, 504, 507, 509, 514,
    516, 517, 518, 520, 524, 525, 532, 538, 540, 541, 542, 543, 544, 551,
    552, 553, 557, 562, 564, 565, 567, 569, 575, 577, 578, 580, 582, 584,
    585, 589, 590, 591, 598, 600, 602, 603, 605, 607, 617, 619, 638, 649,
    650, 654, 659, 670, 673, 675, 681, 690, 693, 694, 698, 703, 704, 706,
    707, 708, 709, 712, 714, 715, 730, 736, 739, 748, 750, 752, 753, 755,
    762, 765, 768, 769, 771, 774, 776, 777, 780, 787, 790, 792, 793, 799,
    803, 804, 808, 810, 816, 829, 836, 842, 846, 848, 854, 857, 859, 864,
    872, 874, 879, 883, 885, 893, 895, 901, 904, 910, 911, 914, 918, 921,
    928, 932, 934, 940, 942, 955, 957, 962, 966, 970, 973, 976, 981, 984,
    995, 996, 999, 1001, 1005, 1009, 1010, 1012, 1016, 1017, 1020, 1021,
], dtype=np.int32)
_MASK = np.zeros((8, _CONCEPT_DIM), dtype=np.float32)
_MASK[:, _IDX] = 1.0

_BLOCK_ROWS = 1024


def _select_kernel(m_ref, x_ref, c_ref, o_ref):
    mask = m_ref[0:1, :] != 0.0
    o_ref[...] = jnp.where(mask, c_ref[...], x_ref[...])


def _tc_kernel_rows(x, concepts, row0, nrows):
    grid = (nrows // _BLOCK_ROWS,)
    r0b = row0 // _BLOCK_ROWS
    in_spec = pl.BlockSpec((_BLOCK_ROWS, _CONCEPT_DIM), lambda i: (i + r0b, 0))
    out_spec = pl.BlockSpec((_BLOCK_ROWS, _CONCEPT_DIM), lambda i: (i, 0))
    mspec = pl.BlockSpec((8, _CONCEPT_DIM), lambda i: (0, 0))
    return pl.pallas_call(
        _select_kernel,
        grid=grid,
        in_specs=[mspec, in_spec, in_spec],
        out_specs=out_spec,
        out_shape=jax.ShapeDtypeStruct((nrows, _CONCEPT_DIM), jnp.float32),
        compiler_params=pltpu.CompilerParams(
            dimension_semantics=("parallel",),
        ),
    )(jnp.asarray(_MASK), x, concepts)


def _tc_kernel(x, concepts):
    return _tc_kernel_rows(x, concepts, 0, _ROWS)


import functools
from jax import lax
from jax.experimental.pallas import tpu_sc as plsc

_LANES = 16
_NW = 32
_ROWS_PER_W = _ROWS // _NW
_CHUNK = 8
_NB = 8
_NCB = 4
_NCHUNK = _ROWS_PER_W // _CHUNK
_NGROUP = _CONCEPT_DIM // _LANES

_MASK1D = np.zeros((_CONCEPT_DIM,), dtype=np.float32)
_MASK1D[_IDX] = 1.0


def _sc_body(m_hbm, x_hbm, c_hbm, o_hbm, mask_v, cbuf, obuf,
             xsem, csem, osem):
    nc = 2
    wid = lax.axis_index("s") * nc + lax.axis_index("c")
    base = wid * _ROWS_PER_W
    pltpu.sync_copy(m_hbm, mask_v)

    def start_in(ch, oslot, cslot):
        row0 = base + ch * _CHUNK
        pltpu.make_async_copy(
            x_hbm.at[pl.ds(row0, _CHUNK)], obuf.at[oslot], xsem.at[oslot]
        ).start()
        pltpu.make_async_copy(
            c_hbm.at[pl.ds(row0, _CHUNK)], cbuf.at[cslot], csem.at[cslot]
        ).start()

    def wait_in(oslot, cslot):
        pltpu.make_async_copy(
            x_hbm.at[pl.ds(base, _CHUNK)], obuf.at[oslot], xsem.at[oslot]
        ).wait()
        pltpu.make_async_copy(
            c_hbm.at[pl.ds(base, _CHUNK)], cbuf.at[cslot], csem.at[cslot]
        ).wait()

    def start_out(ch, oslot):
        row0 = base + ch * _CHUNK
        pltpu.make_async_copy(
            obuf.at[oslot], o_hbm.at[pl.ds(row0, _CHUNK)], osem.at[oslot]
        ).start()

    def wait_out(oslot):
        pltpu.make_async_copy(
            obuf.at[oslot], o_hbm.at[pl.ds(base, _CHUNK)], osem.at[oslot]
        ).wait()

    def compute(oslot, cslot):
        def g_body(g, c2):
            col = pl.multiple_of(g * _LANES, _LANES)
            mg = mask_v[pl.ds(col, _LANES)] != 0.0
            for r in range(_CHUNK):
                xv = obuf[oslot, r, pl.ds(col, _LANES)]
                cv = cbuf[cslot, r, pl.ds(col, _LANES)]
                obuf[oslot, r, pl.ds(col, _LANES)] = jnp.where(mg, cv, xv)
            return c2

        lax.fori_loop(0, _NGROUP, g_body, 0)

    start_in(0, 0, 0)
    start_in(1, 1, 1)

    def ring_body(j, carry):
        for b in range(_NB):
            ch = j * _NB + b
            nxt = ch + 2
            ons = (b + 2) % _NB
            cns = (b + 2) % _NCB

            @pl.when(nxt < _NCHUNK)
            def _():
                @pl.when(nxt >= _NB)
                def _():
                    wait_out(ons)

                start_in(nxt, ons, cns)

            wait_in(b, b % _NCB)
            compute(b, b % _NCB)
            start_out(ch, b)
        return carry

    lax.fori_loop(0, _NCHUNK // _NB, ring_body, 0)
    for b in range(_NB):
        wait_out(b)


def _sc_kernel(x, concepts):
    mesh = plsc.VectorSubcoreMesh(core_axis_name="c", subcore_axis_name="s")
    k = functools.partial(
        pl.kernel,
        mesh=mesh,
        out_type=jax.ShapeDtypeStruct((_ROWS, _CONCEPT_DIM), jnp.float32),
        scratch_types=[
            pltpu.VMEM((_CONCEPT_DIM,), jnp.float32),
            pltpu.VMEM((_NCB, _CHUNK, _CONCEPT_DIM), jnp.float32),
            pltpu.VMEM((_NB, _CHUNK, _CONCEPT_DIM), jnp.float32),
            pltpu.SemaphoreType.DMA((_NB,)),
            pltpu.SemaphoreType.DMA((_NCB,)),
            pltpu.SemaphoreType.DMA((_NB,)),
        ],
    )(_sc_body)
    return k(jnp.asarray(_MASK1D), x, concepts)


def kernel(x, concepts):
    return _sc_kernel(x, concepts)
